# baseline (speedup 1.0000x reference)
import functools

import jax
import jax.numpy as jnp
from jax import lax
from jax.experimental import pallas as pl
from jax.experimental.pallas import tpu as pltpu

N_ROWS = 16384
N_CODES = 8192
DIM = 32

R_BLK = 1024
K_BLK = 1024
N_R = N_ROWS // R_BLK
N_K = N_CODES // K_BLK

_LOSS_SCALE = 1.5 / float(N_ROWS * DIM)
_BIG_I32 = 2**30


_SLAB = 128
_N_SLAB = K_BLK // _SLAB


def _tc_body(x_ref, wt2_ref, idx_ref, loss_ref, rmin_ref, ridx_ref):
    i = pl.program_id(0)
    k = pl.program_id(1)

    xb = x_ref[...]
    wt2 = wt2_ref[...]
    ab2 = lax.dot_general(xb, wt2, (((1,), (0,)), ((), ())),
                          preferred_element_type=jnp.float32)
    a = jnp.sum(xb * xb, axis=1, keepdims=True)
    b = 0.25 * jnp.sum(wt2 * wt2, axis=0, keepdims=True)
    t = a + b

    @pl.when(k == 0)
    def _():
        rmin_ref[...] = jnp.full((R_BLK, _SLAB), jnp.inf, jnp.float32)
        ridx_ref[...] = jnp.zeros((R_BLK, _SLAB), jnp.int32)

    for s in range(_N_SLAB):
        ds = t[:, s * _SLAB:(s + 1) * _SLAB] - ab2[:, s * _SLAB:(s + 1) * _SLAB]
        lt = ds < rmin_ref[...]
        rmin_ref[...] = jnp.where(lt, ds, rmin_ref[...])
        ridx_ref[...] = jnp.where(lt, k * _N_SLAB + s, ridx_ref[...])

    @pl.when(k == N_K - 1)
    def _():
        rmin = rmin_ref[...]
        rowmin = jnp.min(rmin, axis=1, keepdims=True)
        lane = lax.broadcasted_iota(jnp.int32, (R_BLK, _SLAB), 1)
        jmat = ridx_ref[...] * _SLAB + lane
        cidx = jnp.min(jnp.where(rmin == rowmin, jmat, _BIG_I32),
                       axis=1, keepdims=True)
        idx_ref[0] = cidx
        partial = jnp.sum(rowmin)

        @pl.when(i == 0)
        def _():
            loss_ref[0, 0] = partial

        @pl.when(i > 0)
        def _():
            loss_ref[0, 0] = loss_ref[0, 0] + partial

        @pl.when(i == N_R - 1)
        def _():
            loss_ref[0, 0] = loss_ref[0, 0] * _LOSS_SCALE


@functools.lru_cache(maxsize=1)
def _build_tc():
    return pl.pallas_call(
        _tc_body,
        grid=(N_R, N_K),
        in_specs=[
            pl.BlockSpec((R_BLK, DIM), lambda i, k: (i, 0)),
            pl.BlockSpec((DIM, K_BLK), lambda i, k: (0, k)),
        ],
        out_specs=[
            pl.BlockSpec((1, R_BLK, 1), lambda i, k: (i, 0, 0)),
            pl.BlockSpec((1, 1), lambda i, k: (0, 0),
                         memory_space=pltpu.SMEM),
        ],
        out_shape=[
            jax.ShapeDtypeStruct((N_R, R_BLK, 1), jnp.int32),
            jax.ShapeDtypeStruct((1, 1), jnp.float32),
        ],
        scratch_shapes=[
            pltpu.VMEM((R_BLK, _SLAB), jnp.float32),
            pltpu.VMEM((R_BLK, _SLAB), jnp.int32),
        ],
        compiler_params=pltpu.CompilerParams(
            dimension_semantics=("arbitrary", "arbitrary"),
        ),
    )


@functools.lru_cache(maxsize=1)
def _build_sc_gather():
    from jax.experimental.pallas import tpu_sc as plsc

    info = plsc.get_sparse_core_info()
    nc, ns = info.num_cores, info.num_subcores
    nw = nc * ns
    b_per_w = N_ROWS // nw
    n_chunks = b_per_w // 128

    mesh = plsc.VectorSubcoreMesh(core_axis_name="c", subcore_axis_name="s")

    @functools.partial(
        pl.kernel,
        mesh=mesh,
        out_type=jax.ShapeDtypeStruct((N_ROWS, DIM), jnp.float32),
        scratch_types=[
            pltpu.VMEM((n_chunks, 128), jnp.int32),
            pltpu.VMEM((b_per_w, DIM), jnp.float32),
            pltpu.SemaphoreType.DMA,
        ],
        compiler_params=pltpu.CompilerParams(use_tc_tiling_on_sc=False),
    )
    def sc_gather(table_hbm, idx_hbm, out_hbm, idx_v, rows_v, sem):
        wid = lax.axis_index("s") * nc + lax.axis_index("c")
        base = wid * b_per_w
        pltpu.sync_copy(idx_hbm.at[wid], idx_v)
        copies = []
        for j in range(n_chunks):
            copies.append(pltpu.async_copy(
                table_hbm.at[idx_v.at[j]],
                rows_v.at[pl.ds(j * 128, 128)],
                sem))
        for c in copies:
            c.wait()
        pltpu.sync_copy(rows_v, out_hbm.at[pl.ds(base, b_per_w)])

    def run(table, idx_flat):
        nonlocal_shape = (nw, n_chunks, 128)
        return sc_gather(table, idx_flat.reshape(nonlocal_shape))

    return run


def _gather(W, idx_flat):
    return _build_sc_gather()(W, idx_flat)


def kernel(x, W):
    bsz, seq, dim = x.shape
    xf = x.reshape(bsz * seq, dim)
    idx3, loss11 = _build_tc()(xf, W.T * 2.0)
    idx_flat = idx3.reshape(N_ROWS)
    q = _gather(W, idx_flat)
    out = (xf + (q - xf)).reshape(bsz, seq, dim)
    loss = loss11[0, 0]
    return (out, loss)


# speedup vs baseline: 1.9318x; 1.9318x over previous
import functools

import jax
import jax.numpy as jnp
from jax import lax
from jax.experimental import pallas as pl
from jax.experimental.pallas import tpu as pltpu

N_ROWS = 16384
N_CODES = 8192
DIM = 32

R_BLK = 1024
K_BLK = 1024
N_R = N_ROWS // R_BLK
N_K = N_CODES // K_BLK

_LOSS_SCALE = 1.5 / float(N_ROWS * DIM)
_BIG_I32 = 2**30


_SLAB = 128
_N_SLAB = K_BLK // _SLAB


def _tc_body(x_ref, wt2_ref, idx_ref, loss_ref, rmin_ref, ridx_ref):
    i = pl.program_id(0)
    k = pl.program_id(1)

    xb = x_ref[...]
    wt2 = wt2_ref[...]
    ab2 = lax.dot_general(xb, wt2, (((1,), (0,)), ((), ())),
                          preferred_element_type=jnp.float32)
    a = jnp.sum(xb * xb, axis=1, keepdims=True)
    b = 0.25 * jnp.sum(wt2 * wt2, axis=0, keepdims=True)
    t = a + b

    rmin = jnp.where(k == 0,
                     jnp.full((R_BLK, _SLAB), jnp.inf, jnp.float32),
                     rmin_ref[...])
    ridx = ridx_ref[...]
    for s in range(_N_SLAB):
        ds = t[:, s * _SLAB:(s + 1) * _SLAB] - ab2[:, s * _SLAB:(s + 1) * _SLAB]
        lt = ds < rmin
        rmin = jnp.where(lt, ds, rmin)
        ridx = jnp.where(lt, k * _N_SLAB + s, ridx)
    rmin_ref[...] = rmin
    ridx_ref[...] = ridx

    @pl.when(k == N_K - 1)
    def _():
        rmin = rmin_ref[...]
        rowmin = jnp.min(rmin, axis=1, keepdims=True)
        lane = lax.broadcasted_iota(jnp.int32, (R_BLK, _SLAB), 1)
        jmat = ridx_ref[...] * _SLAB + lane
        cidx = jnp.min(jnp.where(rmin == rowmin, jmat, _BIG_I32),
                       axis=1, keepdims=True)
        idx_ref[0] = cidx
        partial = jnp.sum(rowmin)

        @pl.when(i == 0)
        def _():
            loss_ref[0, 0] = partial

        @pl.when(i > 0)
        def _():
            loss_ref[0, 0] = loss_ref[0, 0] + partial

        @pl.when(i == N_R - 1)
        def _():
            loss_ref[0, 0] = loss_ref[0, 0] * _LOSS_SCALE


@functools.lru_cache(maxsize=1)
def _build_tc():
    return pl.pallas_call(
        _tc_body,
        grid=(N_R, N_K),
        in_specs=[
            pl.BlockSpec((R_BLK, DIM), lambda i, k: (i, 0)),
            pl.BlockSpec((DIM, K_BLK), lambda i, k: (0, k)),
        ],
        out_specs=[
            pl.BlockSpec((1, R_BLK, 1), lambda i, k: (i, 0, 0)),
            pl.BlockSpec((1, 1), lambda i, k: (0, 0),
                         memory_space=pltpu.SMEM),
        ],
        out_shape=[
            jax.ShapeDtypeStruct((N_R, R_BLK, 1), jnp.int32),
            jax.ShapeDtypeStruct((1, 1), jnp.float32),
        ],
        scratch_shapes=[
            pltpu.VMEM((R_BLK, _SLAB), jnp.float32),
            pltpu.VMEM((R_BLK, _SLAB), jnp.int32),
        ],
        compiler_params=pltpu.CompilerParams(
            dimension_semantics=("arbitrary", "arbitrary"),
        ),
    )


@functools.lru_cache(maxsize=1)
def _build_sc_gather():
    from jax.experimental.pallas import tpu_sc as plsc

    info = plsc.get_sparse_core_info()
    nc, ns = info.num_cores, info.num_subcores
    nw = nc * ns
    b_per_w = N_ROWS // nw
    n_chunks = b_per_w // 128

    mesh = plsc.VectorSubcoreMesh(core_axis_name="c", subcore_axis_name="s")

    @functools.partial(
        pl.kernel,
        mesh=mesh,
        out_type=jax.ShapeDtypeStruct((N_ROWS, DIM), jnp.float32),
        scratch_types=[
            pltpu.VMEM((n_chunks, 128), jnp.int32),
            pltpu.VMEM((b_per_w, DIM), jnp.float32),
            pltpu.SemaphoreType.DMA,
        ],
        compiler_params=pltpu.CompilerParams(use_tc_tiling_on_sc=False),
    )
    def sc_gather(table_hbm, idx_hbm, out_hbm, idx_v, rows_v, sem):
        wid = lax.axis_index("s") * nc + lax.axis_index("c")
        base = wid * b_per_w
        pltpu.sync_copy(idx_hbm.at[wid], idx_v)
        copies = []
        for j in range(n_chunks):
            copies.append(pltpu.async_copy(
                table_hbm.at[idx_v.at[j]],
                rows_v.at[pl.ds(j * 128, 128)],
                sem))
        for c in copies:
            c.wait()
        pltpu.sync_copy(rows_v, out_hbm.at[pl.ds(base, b_per_w)])

    def run(table, idx_flat):
        nonlocal_shape = (nw, n_chunks, 128)
        return sc_gather(table, idx_flat.reshape(nonlocal_shape))

    return run


def _gather(W, idx_flat):
    return _build_sc_gather()(W, idx_flat)


def kernel(x, W):
    bsz, seq, dim = x.shape
    xf = x.reshape(bsz * seq, dim)
    idx3, loss11 = _build_tc()(xf, W.T * 2.0)
    idx_flat = idx3.reshape(N_ROWS)
    q = _gather(W, idx_flat)
    out = (xf + (q - xf)).reshape(bsz, seq, dim)
    loss = loss11[0, 0]
    return (out, loss)


# speedup vs baseline: 2.0751x; 1.0742x over previous
import functools

import jax
import jax.numpy as jnp
from jax import lax
from jax.experimental import pallas as pl
from jax.experimental.pallas import tpu as pltpu

N_ROWS = 16384
N_CODES = 8192
DIM = 32

R_BLK = 1024
K_BLK = 2048
N_R = N_ROWS // R_BLK
N_K = N_CODES // K_BLK

_LOSS_SCALE = 1.5 / float(N_ROWS * DIM)
_BIG_I32 = 2**30


_SLAB = 128
_N_SLAB = K_BLK // _SLAB


def _tc_body(x_ref, wt2_ref, idx_ref, loss_ref, rmin_ref, ridx_ref):
    i = pl.program_id(0)
    k = pl.program_id(1)

    xb = x_ref[...]
    wt2 = wt2_ref[...]
    ab2 = lax.dot_general(xb, wt2, (((1,), (0,)), ((), ())),
                          preferred_element_type=jnp.float32)
    a = jnp.sum(xb * xb, axis=1, keepdims=True)
    b = 0.25 * jnp.sum(wt2 * wt2, axis=0, keepdims=True)
    t = a + b

    rmin = jnp.where(k == 0,
                     jnp.full((R_BLK, _SLAB), jnp.inf, jnp.float32),
                     rmin_ref[...])
    ridx = ridx_ref[...]
    for s in range(_N_SLAB):
        ds = t[:, s * _SLAB:(s + 1) * _SLAB] - ab2[:, s * _SLAB:(s + 1) * _SLAB]
        lt = ds < rmin
        rmin = jnp.where(lt, ds, rmin)
        ridx = jnp.where(lt, k * _N_SLAB + s, ridx)
    rmin_ref[...] = rmin
    ridx_ref[...] = ridx

    @pl.when(k == N_K - 1)
    def _():
        rmin = rmin_ref[...]
        rowmin = jnp.min(rmin, axis=1, keepdims=True)
        lane = lax.broadcasted_iota(jnp.int32, (R_BLK, _SLAB), 1)
        jmat = ridx_ref[...] * _SLAB + lane
        cidx = jnp.min(jnp.where(rmin == rowmin, jmat, _BIG_I32),
                       axis=1, keepdims=True)
        idx_ref[0] = cidx
        partial = jnp.sum(rowmin)

        @pl.when(i == 0)
        def _():
            loss_ref[0, 0] = partial

        @pl.when(i > 0)
        def _():
            loss_ref[0, 0] = loss_ref[0, 0] + partial

        @pl.when(i == N_R - 1)
        def _():
            loss_ref[0, 0] = loss_ref[0, 0] * _LOSS_SCALE


@functools.lru_cache(maxsize=1)
def _build_tc():
    return pl.pallas_call(
        _tc_body,
        grid=(N_R, N_K),
        in_specs=[
            pl.BlockSpec((R_BLK, DIM), lambda i, k: (i, 0)),
            pl.BlockSpec((DIM, K_BLK), lambda i, k: (0, k)),
        ],
        out_specs=[
            pl.BlockSpec((1, R_BLK, 1), lambda i, k: (i, 0, 0)),
            pl.BlockSpec((1, 1), lambda i, k: (0, 0),
                         memory_space=pltpu.SMEM),
        ],
        out_shape=[
            jax.ShapeDtypeStruct((N_R, R_BLK, 1), jnp.int32),
            jax.ShapeDtypeStruct((1, 1), jnp.float32),
        ],
        scratch_shapes=[
            pltpu.VMEM((R_BLK, _SLAB), jnp.float32),
            pltpu.VMEM((R_BLK, _SLAB), jnp.int32),
        ],
        compiler_params=pltpu.CompilerParams(
            dimension_semantics=("arbitrary", "arbitrary"),
        ),
    )


@functools.lru_cache(maxsize=1)
def _build_sc_gather():
    from jax.experimental.pallas import tpu_sc as plsc

    info = plsc.get_sparse_core_info()
    nc, ns = info.num_cores, info.num_subcores
    nw = nc * ns
    b_per_w = N_ROWS // nw
    n_chunks = b_per_w // 128

    mesh = plsc.VectorSubcoreMesh(core_axis_name="c", subcore_axis_name="s")

    @functools.partial(
        pl.kernel,
        mesh=mesh,
        out_type=jax.ShapeDtypeStruct((N_ROWS, DIM), jnp.float32),
        scratch_types=[
            pltpu.VMEM((n_chunks, 128), jnp.int32),
            pltpu.VMEM((b_per_w, DIM), jnp.float32),
            pltpu.SemaphoreType.DMA,
        ],
        compiler_params=pltpu.CompilerParams(use_tc_tiling_on_sc=False),
    )
    def sc_gather(table_hbm, idx_hbm, out_hbm, idx_v, rows_v, sem):
        wid = lax.axis_index("s") * nc + lax.axis_index("c")
        base = wid * b_per_w
        pltpu.sync_copy(idx_hbm.at[wid], idx_v)
        copies = []
        for j in range(n_chunks):
            copies.append(pltpu.async_copy(
                table_hbm.at[idx_v.at[j]],
                rows_v.at[pl.ds(j * 128, 128)],
                sem))
        for c in copies:
            c.wait()
        pltpu.sync_copy(rows_v, out_hbm.at[pl.ds(base, b_per_w)])

    def run(table, idx_flat):
        nonlocal_shape = (nw, n_chunks, 128)
        return sc_gather(table, idx_flat.reshape(nonlocal_shape))

    return run


def _gather(W, idx_flat):
    return _build_sc_gather()(W, idx_flat)


def kernel(x, W):
    bsz, seq, dim = x.shape
    xf = x.reshape(bsz * seq, dim)
    idx3, loss11 = _build_tc()(xf, W.T * 2.0)
    idx_flat = idx3.reshape(N_ROWS)
    q = _gather(W, idx_flat)
    out = (xf + (q - xf)).reshape(bsz, seq, dim)
    loss = loss11[0, 0]
    return (out, loss)


# speedup vs baseline: 2.0971x; 1.0106x over previous
import functools

import jax
import jax.numpy as jnp
from jax import lax
from jax.experimental import pallas as pl
from jax.experimental.pallas import tpu as pltpu

N_ROWS = 16384
N_CODES = 8192
DIM = 32

R_BLK = 1024
K_BLK = 4096
N_R = N_ROWS // R_BLK
N_K = N_CODES // K_BLK

_LOSS_SCALE = 1.5 / float(N_ROWS * DIM)
_BIG_I32 = 2**30


_SLAB = 128
_N_SLAB = K_BLK // _SLAB


def _tc_body(x_ref, wt2_ref, idx_ref, loss_ref, rmin_ref, ridx_ref):
    i = pl.program_id(0)
    k = pl.program_id(1)

    xb = x_ref[...]
    wt2 = wt2_ref[...]
    ab2 = lax.dot_general(xb, wt2, (((1,), (0,)), ((), ())),
                          preferred_element_type=jnp.float32)
    a = jnp.sum(xb * xb, axis=1, keepdims=True)
    b = 0.25 * jnp.sum(wt2 * wt2, axis=0, keepdims=True)
    t = a + b

    rmin = jnp.where(k == 0,
                     jnp.full((R_BLK, _SLAB), jnp.inf, jnp.float32),
                     rmin_ref[...])
    ridx = ridx_ref[...]
    for s in range(_N_SLAB):
        ds = t[:, s * _SLAB:(s + 1) * _SLAB] - ab2[:, s * _SLAB:(s + 1) * _SLAB]
        lt = ds < rmin
        rmin = jnp.where(lt, ds, rmin)
        ridx = jnp.where(lt, k * _N_SLAB + s, ridx)
    rmin_ref[...] = rmin
    ridx_ref[...] = ridx

    @pl.when(k == N_K - 1)
    def _():
        rmin = rmin_ref[...]
        rowmin = jnp.min(rmin, axis=1, keepdims=True)
        lane = lax.broadcasted_iota(jnp.int32, (R_BLK, _SLAB), 1)
        jmat = ridx_ref[...] * _SLAB + lane
        cidx = jnp.min(jnp.where(rmin == rowmin, jmat, _BIG_I32),
                       axis=1, keepdims=True)
        idx_ref[0] = cidx
        partial = jnp.sum(rowmin)

        @pl.when(i == 0)
        def _():
            loss_ref[0, 0] = partial

        @pl.when(i > 0)
        def _():
            loss_ref[0, 0] = loss_ref[0, 0] + partial

        @pl.when(i == N_R - 1)
        def _():
            loss_ref[0, 0] = loss_ref[0, 0] * _LOSS_SCALE


@functools.lru_cache(maxsize=1)
def _build_tc():
    return pl.pallas_call(
        _tc_body,
        grid=(N_R, N_K),
        in_specs=[
            pl.BlockSpec((R_BLK, DIM), lambda i, k: (i, 0)),
            pl.BlockSpec((DIM, K_BLK), lambda i, k: (0, k)),
        ],
        out_specs=[
            pl.BlockSpec((1, R_BLK, 1), lambda i, k: (i, 0, 0)),
            pl.BlockSpec((1, 1), lambda i, k: (0, 0),
                         memory_space=pltpu.SMEM),
        ],
        out_shape=[
            jax.ShapeDtypeStruct((N_R, R_BLK, 1), jnp.int32),
            jax.ShapeDtypeStruct((1, 1), jnp.float32),
        ],
        scratch_shapes=[
            pltpu.VMEM((R_BLK, _SLAB), jnp.float32),
            pltpu.VMEM((R_BLK, _SLAB), jnp.int32),
        ],
        compiler_params=pltpu.CompilerParams(
            dimension_semantics=("arbitrary", "arbitrary"),
        ),
    )


@functools.lru_cache(maxsize=1)
def _build_sc_gather():
    from jax.experimental.pallas import tpu_sc as plsc

    info = plsc.get_sparse_core_info()
    nc, ns = info.num_cores, info.num_subcores
    nw = nc * ns
    b_per_w = N_ROWS // nw
    n_chunks = b_per_w // 128

    mesh = plsc.VectorSubcoreMesh(core_axis_name="c", subcore_axis_name="s")

    @functools.partial(
        pl.kernel,
        mesh=mesh,
        out_type=jax.ShapeDtypeStruct((N_ROWS, DIM), jnp.float32),
        scratch_types=[
            pltpu.VMEM((n_chunks, 128), jnp.int32),
            pltpu.VMEM((b_per_w, DIM), jnp.float32),
            pltpu.SemaphoreType.DMA,
        ],
        compiler_params=pltpu.CompilerParams(use_tc_tiling_on_sc=False),
    )
    def sc_gather(table_hbm, idx_hbm, out_hbm, idx_v, rows_v, sem):
        wid = lax.axis_index("s") * nc + lax.axis_index("c")
        base = wid * b_per_w
        pltpu.sync_copy(idx_hbm.at[wid], idx_v)
        copies = []
        for j in range(n_chunks):
            copies.append(pltpu.async_copy(
                table_hbm.at[idx_v.at[j]],
                rows_v.at[pl.ds(j * 128, 128)],
                sem))
        for c in copies:
            c.wait()
        pltpu.sync_copy(rows_v, out_hbm.at[pl.ds(base, b_per_w)])

    def run(table, idx_flat):
        nonlocal_shape = (nw, n_chunks, 128)
        return sc_gather(table, idx_flat.reshape(nonlocal_shape))

    return run


def _gather(W, idx_flat):
    return _build_sc_gather()(W, idx_flat)


def kernel(x, W):
    bsz, seq, dim = x.shape
    xf = x.reshape(bsz * seq, dim)
    idx3, loss11 = _build_tc()(xf, W.T * 2.0)
    idx_flat = idx3.reshape(N_ROWS)
    q = _gather(W, idx_flat)
    out = (xf + (q - xf)).reshape(bsz, seq, dim)
    loss = loss11[0, 0]
    return (out, loss)


# speedup vs baseline: 2.2138x; 1.0557x over previous
import functools

import jax
import jax.numpy as jnp
from jax import lax
from jax.experimental import pallas as pl
from jax.experimental.pallas import tpu as pltpu

N_ROWS = 16384
N_CODES = 8192
DIM = 32

R_BLK = 2048
K_BLK = 4096
N_R = N_ROWS // R_BLK
N_K = N_CODES // K_BLK

_LOSS_SCALE = 1.5 / float(N_ROWS * DIM)
_BIG_I32 = 2**30


_SLAB = 128
_N_SLAB = K_BLK // _SLAB


def _tc_body(x_ref, wt2_ref, idx_ref, loss_ref, rmin_ref, ridx_ref):
    i = pl.program_id(0)
    k = pl.program_id(1)

    xb = x_ref[...]
    wt2 = wt2_ref[...]
    ab2 = lax.dot_general(xb, wt2, (((1,), (0,)), ((), ())),
                          preferred_element_type=jnp.float32)
    a = jnp.sum(xb * xb, axis=1, keepdims=True)
    b = 0.25 * jnp.sum(wt2 * wt2, axis=0, keepdims=True)
    t = a + b

    rmin = jnp.where(k == 0,
                     jnp.full((R_BLK, _SLAB), jnp.inf, jnp.float32),
                     rmin_ref[...])
    ridx = ridx_ref[...]
    for s in range(_N_SLAB):
        ds = t[:, s * _SLAB:(s + 1) * _SLAB] - ab2[:, s * _SLAB:(s + 1) * _SLAB]
        lt = ds < rmin
        rmin = jnp.where(lt, ds, rmin)
        ridx = jnp.where(lt, k * _N_SLAB + s, ridx)
    rmin_ref[...] = rmin
    ridx_ref[...] = ridx

    @pl.when(k == N_K - 1)
    def _():
        rmin = rmin_ref[...]
        rowmin = jnp.min(rmin, axis=1, keepdims=True)
        lane = lax.broadcasted_iota(jnp.int32, (R_BLK, _SLAB), 1)
        jmat = ridx_ref[...] * _SLAB + lane
        cidx = jnp.min(jnp.where(rmin == rowmin, jmat, _BIG_I32),
                       axis=1, keepdims=True)
        idx_ref[0] = cidx
        partial = jnp.sum(rowmin)

        @pl.when(i == 0)
        def _():
            loss_ref[0, 0] = partial

        @pl.when(i > 0)
        def _():
            loss_ref[0, 0] = loss_ref[0, 0] + partial

        @pl.when(i == N_R - 1)
        def _():
            loss_ref[0, 0] = loss_ref[0, 0] * _LOSS_SCALE


@functools.lru_cache(maxsize=1)
def _build_tc():
    return pl.pallas_call(
        _tc_body,
        grid=(N_R, N_K),
        in_specs=[
            pl.BlockSpec((R_BLK, DIM), lambda i, k: (i, 0)),
            pl.BlockSpec((DIM, K_BLK), lambda i, k: (0, k)),
        ],
        out_specs=[
            pl.BlockSpec((1, R_BLK, 1), lambda i, k: (i, 0, 0)),
            pl.BlockSpec((1, 1), lambda i, k: (0, 0),
                         memory_space=pltpu.SMEM),
        ],
        out_shape=[
            jax.ShapeDtypeStruct((N_R, R_BLK, 1), jnp.int32),
            jax.ShapeDtypeStruct((1, 1), jnp.float32),
        ],
        scratch_shapes=[
            pltpu.VMEM((R_BLK, _SLAB), jnp.float32),
            pltpu.VMEM((R_BLK, _SLAB), jnp.int32),
        ],
        compiler_params=pltpu.CompilerParams(
            dimension_semantics=("arbitrary", "arbitrary"),
        ),
    )


@functools.lru_cache(maxsize=1)
def _build_sc_gather():
    from jax.experimental.pallas import tpu_sc as plsc

    info = plsc.get_sparse_core_info()
    nc, ns = info.num_cores, info.num_subcores
    nw = nc * ns
    b_per_w = N_ROWS // nw
    n_chunks = b_per_w // 128

    mesh = plsc.VectorSubcoreMesh(core_axis_name="c", subcore_axis_name="s")

    @functools.partial(
        pl.kernel,
        mesh=mesh,
        out_type=jax.ShapeDtypeStruct((N_ROWS, DIM), jnp.float32),
        scratch_types=[
            pltpu.VMEM((n_chunks, 128), jnp.int32),
            pltpu.VMEM((b_per_w, DIM), jnp.float32),
            pltpu.SemaphoreType.DMA,
        ],
        compiler_params=pltpu.CompilerParams(use_tc_tiling_on_sc=False),
    )
    def sc_gather(table_hbm, idx_hbm, out_hbm, idx_v, rows_v, sem):
        wid = lax.axis_index("s") * nc + lax.axis_index("c")
        base = wid * b_per_w
        pltpu.sync_copy(idx_hbm.at[wid], idx_v)
        copies = []
        for j in range(n_chunks):
            copies.append(pltpu.async_copy(
                table_hbm.at[idx_v.at[j]],
                rows_v.at[pl.ds(j * 128, 128)],
                sem))
        for c in copies:
            c.wait()
        pltpu.sync_copy(rows_v, out_hbm.at[pl.ds(base, b_per_w)])

    def run(table, idx_flat):
        nonlocal_shape = (nw, n_chunks, 128)
        return sc_gather(table, idx_flat.reshape(nonlocal_shape))

    return run


def _gather(W, idx_flat):
    return _build_sc_gather()(W, idx_flat)


def kernel(x, W):
    bsz, seq, dim = x.shape
    xf = x.reshape(bsz * seq, dim)
    idx3, loss11 = _build_tc()(xf, W.T * 2.0)
    idx_flat = idx3.reshape(N_ROWS)
    q = _gather(W, idx_flat)
    out = (xf + (q - xf)).reshape(bsz, seq, dim)
    loss = loss11[0, 0]
    return (out, loss)
